# Initial kernel scaffold; baseline (speedup 1.0000x reference)
import jax
import jax.numpy as jnp
from jax.experimental import pallas as pl


def kernel(xyz, params):
    raise NotImplementedError("write your pallas kernel here")


# speedup vs baseline: 2.2252x; 2.2252x over previous
import jax
import jax.numpy as jnp
from jax.experimental import pallas as pl
from jax.experimental.pallas import tpu as pltpu


def _fps_kernel(x_ref, out_ref):
    n = x_ref.shape[1]
    n_center = out_ref.shape[1]
    x = x_ref[...]
    iota = jax.lax.broadcasted_iota(jnp.int32, (1, n), 1)

    def body(i, carry):
        dist, far = carry
        out_ref[0, i] = far
        mask = iota == far
        c = jnp.sum(jnp.where(mask, x, 0.0), axis=1, keepdims=True)
        d = jnp.sum((x - c) ** 2, axis=0, keepdims=True)
        dist = jnp.minimum(dist, d)
        far = jnp.argmax(dist).astype(jnp.int32)
        return dist, far

    dist0 = jnp.full((1, n), 1e10, jnp.float32)
    jax.lax.fori_loop(0, n_center, body, (dist0, jnp.int32(0)))


def fps(xyz, n_center):
    x = jax.lax.stop_gradient(xyz)
    b, n, _ = x.shape
    xt = jnp.transpose(x, (0, 2, 1))
    outs = []
    for bi in range(b):
        out = pl.pallas_call(
            _fps_kernel,
            out_shape=jax.ShapeDtypeStruct((1, n_center), jnp.int32),
            in_specs=[pl.BlockSpec(memory_space=pltpu.VMEM)],
            out_specs=pl.BlockSpec(memory_space=pltpu.SMEM),
        )(xt[bi])
        outs.append(out)
    return jnp.concatenate(outs, axis=0)


def square_distance(src, dst):
    return (jnp.sum(src ** 2, -1)[:, :, None] + jnp.sum(dst ** 2, -1)[:, None, :]
            - 2.0 * jnp.einsum('bnc,bmc->bnm', src, dst))


def index_points(points, idx):
    return jax.vmap(lambda p, i: p[i])(points, idx)


def surface_knn(xyz, n_near):
    x = jax.lax.stop_gradient(xyz)
    d = square_distance(x, x)
    _, idx = jax.lax.top_k(-d, n_near)
    return idx


def mlp(x, p):
    h = x.transpose(0, 2, 1)
    nl = len(p['W'])
    for i in range(nl):
        h = h @ p['W'][i] + p['b'][i]
        if i < nl - 1:
            h = jax.nn.relu(h)
    return h.transpose(0, 2, 1)


def point_attention(center_fea, g_fea, p):
    q = center_fea @ p['Wq']
    k = g_fea @ p['Wk']
    v = g_fea @ p['Wv']
    logits = jnp.einsum('bnc,bnkc->bnk', q, k) / jnp.sqrt(jnp.float32(q.shape[-1]))
    attn = jax.nn.softmax(logits, axis=-1)
    return jnp.einsum('bnk,bnkc->bnc', attn, v)


def sa_layer(xyz_cn, fea_cn, p, n_center, n_near):
    xyz = xyz_cn.transpose(0, 2, 1)
    fea = fea_cn.transpose(0, 2, 1)
    idx_all = surface_knn(xyz, n_near)
    fps_idx = fps(xyz, n_center)
    idx = index_points(idx_all, fps_idx)
    center_xyz = index_points(xyz, fps_idx)
    g_xyz = index_points(xyz, idx)
    xyz_rel = g_xyz - center_xyz[:, :, None, :]
    center_fea = index_points(fea, fps_idx)
    g_fea = index_points(fea, idx)
    g_fea = jnp.concatenate([g_fea, xyz_rel], axis=-1)
    new_fea = point_attention(center_fea, g_fea, p)
    return center_xyz.transpose(0, 2, 1), new_fea.transpose(0, 2, 1)


def upsample(xyz1_cn, xyz2_cn, points1_cn, points2_cn, p):
    xyz1 = xyz1_cn.transpose(0, 2, 1)
    xyz2 = xyz2_cn.transpose(0, 2, 1)
    points2 = points2_cn.transpose(0, 2, 1)
    d = square_distance(xyz1, xyz2)
    neg_vals, idx = jax.lax.top_k(-d, 3)
    dists = -neg_vals
    dist_recip = 1.0 / (dists + 1e-8)
    norm = jnp.sum(dist_recip, axis=2, keepdims=True)
    weight = dist_recip / norm
    interpolated = jnp.sum(index_points(points2, idx) * weight[..., None], axis=2)
    points1 = points1_cn.transpose(0, 2, 1)
    new_points = jnp.concatenate([points1, interpolated], axis=-1).transpose(0, 2, 1)
    return mlp(new_points, p)


def kernel(xyz, params):
    n_points = xyz.shape[1]
    drate = 0.9
    n1 = int(n_points * drate)
    n2 = int(n_points * drate ** 2)
    n3 = int(n_points * drate ** 3)
    x0 = xyz.transpose(0, 2, 1)
    l1_xyz, l1_points = sa_layer(x0, x0, params['sa1'], n1, 50)
    l2_xyz, l2_points = sa_layer(l1_xyz, l1_points, params['sa2'], n2, 75)
    l3_xyz, l3_points = sa_layer(l2_xyz, l2_points, params['sa3'], n3, 100)
    l2_points = upsample(l2_xyz, l3_xyz, l2_points, l3_points, params['fp3'])
    l1_points = upsample(l1_xyz, l2_xyz, l1_points, l2_points, params['fp2'])
    l0_points = upsample(x0, l1_xyz, jnp.concatenate([x0, x0], axis=1), l1_points, params['fp1'])
    feat = mlp(l0_points, params['mlp_fea'])
    mad = mlp(feat, params['mlp_mad']).transpose(0, 2, 1)
    adj = mlp(feat, params['mlp_adj']).transpose(0, 2, 1)
    pt = mlp(feat, params['mlp_pt']).transpose(0, 2, 1)
    return mad, jax.nn.log_softmax(adj, axis=-1), jax.nn.log_softmax(pt, axis=-1)
